# Initial kernel scaffold; baseline (speedup 1.0000x reference)
import jax
import jax.numpy as jnp
from jax.experimental import pallas as pl


def kernel(x, y, usr_embd, usr_bias, mov_embd, mov_bias, fc_W, fc_b):
    raise NotImplementedError("write your pallas kernel here")


# speedup vs baseline: 1.4221x; 1.4221x over previous
import functools

import jax
import jax.numpy as jnp
from jax import lax
from jax.experimental import pallas as pl
from jax.experimental.pallas import tpu as pltpu
from jax.experimental.pallas import tpu_sc as plsc

B = 16384
EMB = 16
L = 16
NC = 2
NS = 16
J_PER = B // NS
I_PER = B // (NC * NS)
CH = 128


def _body(usr_embd, usr_bias, mov_embd, mov_bias, xv, yv, fcw, fcb,
          out_hbm,
          yidx, xidx, yidx2, me_buf, ue_buf, ub_buf, mbi_buf, w_buf, out_buf,
          fcb_buf, red_buf, stage_buf, shared, sem_mov, sem_usr):
    c = lax.axis_index("c")
    s = lax.axis_index("s")
    wid = s * NC + c
    jbase = s * J_PER
    ibase = wid * I_PER

    mov_copies = []
    usr_copies = []
    for i in range(J_PER // CH):
        pltpu.sync_copy(yv.at[pl.ds(jbase + i * CH, CH)], yidx.at[i])
        mov_copies.append(pltpu.async_copy(
            mov_embd.at[yidx.at[i]], me_buf.at[pl.ds(i * CH, CH)], sem_mov))
    for i in range(I_PER // CH):
        pltpu.sync_copy(xv.at[pl.ds(ibase + i * CH, CH)], xidx.at[i])
        pltpu.sync_copy(yv.at[pl.ds(ibase + i * CH, CH)], yidx2.at[i])
        usr_copies.append(pltpu.async_copy(
            usr_embd.at[xidx.at[i]], ue_buf.at[pl.ds(i * CH, CH)], sem_usr))
        usr_copies.append(pltpu.async_copy(
            usr_bias.at[xidx.at[i]], ub_buf.at[pl.ds(i * CH, CH)], sem_usr))
        usr_copies.append(pltpu.async_copy(
            mov_bias.at[yidx2.at[i]], mbi_buf.at[pl.ds(i * CH, CH)], sem_usr))
    pltpu.sync_copy(fcw.at[pl.ds(jbase, J_PER)], w_buf)
    pltpu.sync_copy(fcb, fcb_buf)
    for cp in mov_copies:
        cp.wait()

    zero = jnp.zeros((L,), jnp.float32)
    iota = lax.iota(jnp.int32, L)

    def p1_body(b, carry):
        vacc, wacc = carry
        wacc = wacc + w_buf[pl.ds(b * L, L)]
        for k in range(L):
            j = b * L + k
            wb = plsc.load_gather(w_buf, [jnp.full((L,), j, jnp.int32)])
            vacc = vacc + wb * me_buf[j]
        return (vacc, wacc)

    vacc, wacc = lax.fori_loop(0, J_PER // L, p1_body, (zero, zero))

    stage_buf[pl.ds(0, L)] = vacc
    stage_buf[pl.ds(L, L)] = wacc
    pltpu.sync_copy(stage_buf, shared.at[s])
    plsc.subcore_barrier()
    pltpu.sync_copy(shared, red_buf)
    v = zero
    wv = zero
    for t in range(NS):
        v = v + red_buf[t, pl.ds(0, L)]
        wv = wv + red_buf[t, pl.ds(L, L)]
    fcb_splat = fcb_buf[...]
    w_splat = zero + jnp.sum(wv)
    vs = [zero + jnp.sum(jnp.where(iota == d, v, zero)) for d in range(L)]

    for cp in usr_copies:
        cp.wait()

    def p2_body(b, carry):
        bias = ub_buf[pl.ds(b * L, L)] + mbi_buf[pl.ds(b * L, L)]
        acc = fcb_splat + bias * w_splat
        rows = b * L + iota
        for d in range(L):
            col = plsc.load_gather(ue_buf, [rows, jnp.full((L,), d, jnp.int32)])
            acc = acc + col * vs[d]
        out_buf[pl.ds(b * L, L)] = 1.0 / (1.0 + jnp.exp(-acc))
        return carry

    lax.fori_loop(0, I_PER // L, p2_body, 0)
    pltpu.sync_copy(out_buf, out_hbm.at[pl.ds(ibase, I_PER)])


@jax.jit
def kernel(x, y, usr_embd, usr_bias, mov_embd, mov_bias, fc_W, fc_b):
    mesh = plsc.VectorSubcoreMesh(core_axis_name="c", subcore_axis_name="s")
    run = functools.partial(
        pl.kernel,
        out_type=jax.ShapeDtypeStruct((B,), jnp.float32),
        mesh=mesh,
        compiler_params=pltpu.CompilerParams(
            needs_layout_passes=False, use_tc_tiling_on_sc=False),
        scratch_types=[
            pltpu.VMEM((J_PER // CH, CH), jnp.int32),
            pltpu.VMEM((I_PER // CH, CH), jnp.int32),
            pltpu.VMEM((I_PER // CH, CH), jnp.int32),
            pltpu.VMEM((J_PER, EMB), jnp.float32),
            pltpu.VMEM((I_PER, EMB), jnp.float32),
            pltpu.VMEM((I_PER,), jnp.float32),
            pltpu.VMEM((I_PER,), jnp.float32),
            pltpu.VMEM((J_PER,), jnp.float32),
            pltpu.VMEM((I_PER,), jnp.float32),
            pltpu.VMEM((L,), jnp.float32),
            pltpu.VMEM((NS, 2 * L), jnp.float32),
            pltpu.VMEM((2 * L,), jnp.float32),
            pltpu.VMEM_SHARED((NS, 2 * L), jnp.float32),
            pltpu.SemaphoreType.DMA,
            pltpu.SemaphoreType.DMA,
        ],
    )(_body)
    out = run(usr_embd, usr_bias.reshape(-1), mov_embd, mov_bias.reshape(-1),
              x.astype(jnp.int32), y.astype(jnp.int32),
              fc_W.reshape(-1), jnp.broadcast_to(fc_b, (L,)))
    return out.reshape(B, 1)


# speedup vs baseline: 4.8561x; 3.4147x over previous
import functools

import jax
import jax.numpy as jnp
from jax import lax
from jax.experimental import pallas as pl
from jax.experimental.pallas import tpu as pltpu
from jax.experimental.pallas import tpu_sc as plsc

B = 16384
EMB = 16
L = 16
NC = 2
NS = 16
J_PER = B // NS
I_PER = B // (NC * NS)
CH = 128

N_USR = 1000001
BLK = 65536
GRID = -(-N_USR // BLK)
N_PAD = GRID * BLK


def _sc_movie(mov_embd, mov_bias, yv, fcw,
              vw_out, bw_out,
              yidx, yidx2, me_buf, mbi_buf, w_buf, bw_buf, vw_buf,
              red_buf, stage_buf, shared, sem_me, sem_mb):
    c = lax.axis_index("c")
    s = lax.axis_index("s")
    wid = s * NC + c
    jbase = s * J_PER
    ibase = wid * I_PER

    me_copies = []
    mb_copies = []
    for i in range(J_PER // CH):
        pltpu.sync_copy(yv.at[pl.ds(jbase + i * CH, CH)], yidx.at[i])
        me_copies.append(pltpu.async_copy(
            mov_embd.at[yidx.at[i]], me_buf.at[pl.ds(i * CH, CH)], sem_me))
    for i in range(I_PER // CH):
        pltpu.sync_copy(yv.at[pl.ds(ibase + i * CH, CH)], yidx2.at[i])
        mb_copies.append(pltpu.async_copy(
            mov_bias.at[yidx2.at[i]], mbi_buf.at[pl.ds(i * CH, CH)], sem_mb))
    pltpu.sync_copy(fcw.at[pl.ds(jbase, J_PER)], w_buf)
    for cp in me_copies:
        cp.wait()

    zero = jnp.zeros((L,), jnp.float32)
    iota = lax.iota(jnp.int32, L)

    def p1_body(b, carry):
        vacc, wacc = carry
        wacc = wacc + w_buf[pl.ds(b * L, L)]
        for k in range(L):
            j = b * L + k
            wb = plsc.load_gather(w_buf, [jnp.full((L,), j, jnp.int32)])
            vacc = vacc + wb * me_buf[j]
        return (vacc, wacc)

    vacc, wacc = lax.fori_loop(0, J_PER // L, p1_body, (zero, zero))

    stage_buf[pl.ds(0, L)] = vacc
    stage_buf[pl.ds(L, L)] = wacc
    pltpu.sync_copy(stage_buf, shared.at[s])
    plsc.subcore_barrier()
    pltpu.sync_copy(shared, red_buf)
    v = zero
    wv = zero
    for t in range(NS):
        v = v + red_buf[t, pl.ds(0, L)]
        wv = wv + red_buf[t, pl.ds(L, L)]
    w_splat = zero + jnp.sum(wv)

    for cp in mb_copies:
        cp.wait()

    def bw_body(b, carry):
        bw_buf[pl.ds(b * L, L)] = w_splat * mbi_buf[pl.ds(b * L, L)]
        return carry

    lax.fori_loop(0, I_PER // L, bw_body, 0)
    pltpu.sync_copy(bw_buf, bw_out.at[pl.ds(ibase, I_PER)])

    @pl.when(wid == 0)
    def _():
        vw_buf[pl.ds(0, L)] = v
        vw_buf[pl.ds(L, L)] = w_splat
        vw_buf[pl.ds(2 * L, L)] = zero
        vw_buf[pl.ds(3 * L, L)] = zero
        vw_buf[pl.ds(4 * L, L)] = zero
        vw_buf[pl.ds(5 * L, L)] = zero
        vw_buf[pl.ds(6 * L, L)] = zero
        vw_buf[pl.ds(7 * L, L)] = zero
        pltpu.sync_copy(vw_buf, vw_out.at[0])


def _tc_user(ue_ref, ub_ref, vw_ref, out_ref):
    lhs = vw_ref[0:1, 0:EMB]
    u = jax.lax.dot_general(lhs, ue_ref[...],
                            (((1,), (0,)), ((), ())),
                            precision=jax.lax.Precision.HIGHEST)
    out_ref[...] = (u + vw_ref[0, EMB] * ub_ref[...])[0]


def _sc_out(u_hbm, bw_hbm, xv, fcb,
            out_hbm,
            xidx, u_buf, bwi_buf, fcb_buf, out_buf, sem):
    c = lax.axis_index("c")
    s = lax.axis_index("s")
    wid = s * NC + c
    ibase = wid * I_PER

    copies = []
    for i in range(I_PER // CH):
        pltpu.sync_copy(xv.at[pl.ds(ibase + i * CH, CH)], xidx.at[i])
        copies.append(pltpu.async_copy(
            u_hbm.at[xidx.at[i]], u_buf.at[pl.ds(i * CH, CH)], sem))
    pltpu.sync_copy(bw_hbm.at[pl.ds(ibase, I_PER)], bwi_buf)
    pltpu.sync_copy(fcb, fcb_buf)
    for cp in copies:
        cp.wait()
    fcb_splat = fcb_buf[...]

    def body(b, carry):
        z = u_buf[pl.ds(b * L, L)] + bwi_buf[pl.ds(b * L, L)] + fcb_splat
        out_buf[pl.ds(b * L, L)] = 1.0 / (1.0 + jnp.exp(-z))
        return carry

    lax.fori_loop(0, I_PER // L, body, 0)
    pltpu.sync_copy(out_buf, out_hbm.at[pl.ds(ibase, I_PER)])


@jax.jit
def kernel(x, y, usr_embd, usr_bias, mov_embd, mov_bias, fc_W, fc_b):
    xi = x.astype(jnp.int32)
    yi = y.astype(jnp.int32)
    mesh = plsc.VectorSubcoreMesh(core_axis_name="c", subcore_axis_name="s")
    sc_params = pltpu.CompilerParams(
        needs_layout_passes=False, use_tc_tiling_on_sc=False)

    movie = functools.partial(
        pl.kernel,
        out_type=(jax.ShapeDtypeStruct((1, 128), jnp.float32),
                  jax.ShapeDtypeStruct((B,), jnp.float32)),
        mesh=mesh,
        compiler_params=sc_params,
        scratch_types=[
            pltpu.VMEM((J_PER // CH, CH), jnp.int32),
            pltpu.VMEM((I_PER // CH, CH), jnp.int32),
            pltpu.VMEM((J_PER, EMB), jnp.float32),
            pltpu.VMEM((I_PER,), jnp.float32),
            pltpu.VMEM((J_PER,), jnp.float32),
            pltpu.VMEM((I_PER,), jnp.float32),
            pltpu.VMEM((128,), jnp.float32),
            pltpu.VMEM((NS, 2 * L), jnp.float32),
            pltpu.VMEM((2 * L,), jnp.float32),
            pltpu.VMEM_SHARED((NS, 2 * L), jnp.float32),
            pltpu.SemaphoreType.DMA,
            pltpu.SemaphoreType.DMA,
        ],
    )(_sc_movie)
    vw, bw = movie(mov_embd, mov_bias.reshape(-1), yi, fc_W.reshape(-1))

    u_all = pl.pallas_call(
        _tc_user,
        grid=(GRID,),
        in_specs=[
            pl.BlockSpec((EMB, BLK), lambda i: (0, i)),
            pl.BlockSpec((1, BLK), lambda i: (0, i)),
            pl.BlockSpec((1, 128), lambda i: (0, 0)),
        ],
        out_specs=pl.BlockSpec((BLK,), lambda i: (i,)),
        out_shape=jax.ShapeDtypeStruct((N_PAD,), jnp.float32),
    )(usr_embd.T, usr_bias.T, vw)

    final = functools.partial(
        pl.kernel,
        out_type=jax.ShapeDtypeStruct((B,), jnp.float32),
        mesh=mesh,
        compiler_params=sc_params,
        scratch_types=[
            pltpu.VMEM((I_PER // CH, CH), jnp.int32),
            pltpu.VMEM((I_PER,), jnp.float32),
            pltpu.VMEM((I_PER,), jnp.float32),
            pltpu.VMEM((L,), jnp.float32),
            pltpu.VMEM((I_PER,), jnp.float32),
            pltpu.SemaphoreType.DMA,
        ],
    )(_sc_out)
    out = final(u_all, bw, xi, jnp.broadcast_to(fc_b, (L,)))
    return out.reshape(B, 1)


# speedup vs baseline: 6.4853x; 1.3355x over previous
import functools

import jax
import jax.numpy as jnp
from jax import lax
from jax.experimental import pallas as pl
from jax.experimental.pallas import tpu as pltpu
from jax.experimental.pallas import tpu_sc as plsc

B = 16384
EMB = 16
L = 16
NC = 2
NS = 16
J_PER = B // NS
I_PER = B // (NC * NS)
CH = 128

N_USR = 1000001
BLK = 65536
GRID = -(-N_USR // BLK)
N_PAD = GRID * BLK


def _sc_movie(mov_embd, mov_bias, yv, fcw,
              vw_out, bw_out,
              yidx, yidx2, me_buf, mbi_buf, w_buf, bw_buf, vw_buf,
              red_buf, stage_buf, shared, sem_me, sem_mb):
    c = lax.axis_index("c")
    s = lax.axis_index("s")
    wid = s * NC + c
    jbase = s * J_PER
    ibase = wid * I_PER

    me_copies = []
    mb_copies = []
    for i in range(J_PER // CH):
        pltpu.sync_copy(yv.at[pl.ds(jbase + i * CH, CH)], yidx.at[i])
        me_copies.append(pltpu.async_copy(
            mov_embd.at[yidx.at[i]], me_buf.at[pl.ds(i * CH, CH)], sem_me))
    for i in range(I_PER // CH):
        pltpu.sync_copy(yv.at[pl.ds(ibase + i * CH, CH)], yidx2.at[i])
        mb_copies.append(pltpu.async_copy(
            mov_bias.at[yidx2.at[i]], mbi_buf.at[pl.ds(i * CH, CH)], sem_mb))
    pltpu.sync_copy(fcw.at[pl.ds(jbase, J_PER)], w_buf)
    for cp in me_copies:
        cp.wait()

    zero = jnp.zeros((L,), jnp.float32)
    iota = lax.iota(jnp.int32, L)

    def p1_body(b, carry):
        vacc, wacc = carry
        wacc = wacc + w_buf[pl.ds(b * L, L)]
        for k in range(L):
            j = b * L + k
            wb = plsc.load_gather(w_buf, [jnp.full((L,), j, jnp.int32)])
            vacc = vacc + wb * me_buf[j]
        return (vacc, wacc)

    vacc, wacc = lax.fori_loop(0, J_PER // L, p1_body, (zero, zero))

    stage_buf[pl.ds(0, L)] = vacc
    stage_buf[pl.ds(L, L)] = wacc
    pltpu.sync_copy(stage_buf, shared.at[s])
    plsc.subcore_barrier()
    pltpu.sync_copy(shared, red_buf)
    v = zero
    wv = zero
    for t in range(NS):
        v = v + red_buf[t, pl.ds(0, L)]
        wv = wv + red_buf[t, pl.ds(L, L)]
    w_splat = zero + jnp.sum(wv)

    for cp in mb_copies:
        cp.wait()

    def bw_body(b, carry):
        bw_buf[pl.ds(b * L, L)] = w_splat * mbi_buf[pl.ds(b * L, L)]
        return carry

    lax.fori_loop(0, I_PER // L, bw_body, 0)
    pltpu.sync_copy(bw_buf, bw_out.at[pl.ds(ibase, I_PER)])

    @pl.when(wid == 0)
    def _():
        vw_buf[pl.ds(0, L)] = v
        vw_buf[pl.ds(L, L)] = w_splat
        vw_buf[pl.ds(2 * L, L)] = zero
        vw_buf[pl.ds(3 * L, L)] = zero
        vw_buf[pl.ds(4 * L, L)] = zero
        vw_buf[pl.ds(5 * L, L)] = zero
        vw_buf[pl.ds(6 * L, L)] = zero
        vw_buf[pl.ds(7 * L, L)] = zero
        pltpu.sync_copy(vw_buf, vw_out.at[0])


def _tc_user(ue_ref, ub_ref, vw_ref, out_ref):
    v_col = vw_ref[0, 0:EMB][:, None]
    u = jnp.sum(ue_ref[...] * v_col, axis=0)
    out_ref[...] = u + vw_ref[0, EMB] * ub_ref[0, :]


def _sc_out(u_hbm, bw_hbm, xv, fcb,
            out_hbm,
            xidx, u_buf, bwi_buf, fcb_buf, out_buf, sem):
    c = lax.axis_index("c")
    s = lax.axis_index("s")
    wid = s * NC + c
    ibase = wid * I_PER

    copies = []
    for i in range(I_PER // CH):
        pltpu.sync_copy(xv.at[pl.ds(ibase + i * CH, CH)], xidx.at[i])
        copies.append(pltpu.async_copy(
            u_hbm.at[xidx.at[i]], u_buf.at[pl.ds(i * CH, CH)], sem))
    pltpu.sync_copy(bw_hbm.at[pl.ds(ibase, I_PER)], bwi_buf)
    pltpu.sync_copy(fcb, fcb_buf)
    for cp in copies:
        cp.wait()
    fcb_splat = fcb_buf[...]

    def body(b, carry):
        z = u_buf[pl.ds(b * L, L)] + bwi_buf[pl.ds(b * L, L)] + fcb_splat
        out_buf[pl.ds(b * L, L)] = 1.0 / (1.0 + jnp.exp(-z))
        return carry

    lax.fori_loop(0, I_PER // L, body, 0)
    pltpu.sync_copy(out_buf, out_hbm.at[pl.ds(ibase, I_PER)])


@jax.jit
def kernel(x, y, usr_embd, usr_bias, mov_embd, mov_bias, fc_W, fc_b):
    xi = x.astype(jnp.int32)
    yi = y.astype(jnp.int32)
    mesh = plsc.VectorSubcoreMesh(core_axis_name="c", subcore_axis_name="s")
    sc_params = pltpu.CompilerParams(
        needs_layout_passes=False, use_tc_tiling_on_sc=False)

    movie = functools.partial(
        pl.kernel,
        out_type=(jax.ShapeDtypeStruct((1, 128), jnp.float32),
                  jax.ShapeDtypeStruct((B,), jnp.float32)),
        mesh=mesh,
        compiler_params=sc_params,
        scratch_types=[
            pltpu.VMEM((J_PER // CH, CH), jnp.int32),
            pltpu.VMEM((I_PER // CH, CH), jnp.int32),
            pltpu.VMEM((J_PER, EMB), jnp.float32),
            pltpu.VMEM((I_PER,), jnp.float32),
            pltpu.VMEM((J_PER,), jnp.float32),
            pltpu.VMEM((I_PER,), jnp.float32),
            pltpu.VMEM((128,), jnp.float32),
            pltpu.VMEM((NS, 2 * L), jnp.float32),
            pltpu.VMEM((2 * L,), jnp.float32),
            pltpu.VMEM_SHARED((NS, 2 * L), jnp.float32),
            pltpu.SemaphoreType.DMA,
            pltpu.SemaphoreType.DMA,
        ],
    )(_sc_movie)
    vw, bw = movie(mov_embd, mov_bias.reshape(-1), yi, fc_W.reshape(-1))

    u_all = pl.pallas_call(
        _tc_user,
        grid=(GRID,),
        in_specs=[
            pl.BlockSpec((EMB, BLK), lambda i: (0, i)),
            pl.BlockSpec((1, BLK), lambda i: (0, i)),
            pl.BlockSpec((1, 128), lambda i: (0, 0)),
        ],
        out_specs=pl.BlockSpec((BLK,), lambda i: (i,)),
        out_shape=jax.ShapeDtypeStruct((N_PAD,), jnp.float32),
    )(usr_embd.T, usr_bias.T, vw)

    final = functools.partial(
        pl.kernel,
        out_type=jax.ShapeDtypeStruct((B,), jnp.float32),
        mesh=mesh,
        compiler_params=sc_params,
        scratch_types=[
            pltpu.VMEM((I_PER // CH, CH), jnp.int32),
            pltpu.VMEM((I_PER,), jnp.float32),
            pltpu.VMEM((I_PER,), jnp.float32),
            pltpu.VMEM((L,), jnp.float32),
            pltpu.VMEM((I_PER,), jnp.float32),
            pltpu.SemaphoreType.DMA,
        ],
    )(_sc_out)
    out = final(u_all, bw, xi, jnp.broadcast_to(fc_b, (L,)))
    return out.reshape(B, 1)


# speedup vs baseline: 10.1283x; 1.5617x over previous
import functools

import jax
import jax.numpy as jnp
from jax import lax
from jax.experimental import pallas as pl
from jax.experimental.pallas import tpu as pltpu
from jax.experimental.pallas import tpu_sc as plsc

B = 16384
EMB = 16
L = 16
NC = 2
NS = 16
I_PER = B // (NC * NS)
CH = 128

N_USR = 1000001
N_MOV = 100001
NSEG = 6256
N_MOV_PAD = NS * NSEG
BLK = 65536
GRID = -(-N_USR // BLK)
N_PAD = GRID * BLK


def _sc_scatter(yv, fcw, s_out,
                yidx, w_buf, zero_buf, acc):
    c = lax.axis_index("c")
    s = lax.axis_index("s")
    wid = s * NC + c
    jbase = wid * I_PER

    zero = jnp.zeros((L,), jnp.float32)

    def zbody(b, carry):
        zero_buf[pl.ds(b * L, L)] = zero
        return carry

    lax.fori_loop(0, NSEG // L, zbody, 0)
    pltpu.sync_copy(zero_buf, acc.at[pl.ds(s * NSEG, NSEG)])
    plsc.subcore_barrier()

    for i in range(I_PER // CH):
        pltpu.sync_copy(yv.at[pl.ds(jbase + i * CH, CH)], yidx.at[i])
        pltpu.sync_copy(fcw.at[pl.ds(jbase + i * CH, CH)], w_buf.at[i])
        pltpu.sync_copy(w_buf.at[i], acc.at[yidx.at[i]], add=True)
    plsc.subcore_barrier()
    pltpu.sync_copy(acc.at[pl.ds(s * NSEG, NSEG)],
                    s_out.at[pl.ds(c * N_MOV_PAD + s * NSEG, NSEG)])


def _tc_user(ue_ref, ub_ref, s_ref, me_ref, out_ref, vw_out, vw_scr):
    @pl.when(pl.program_id(0) == 0)
    def _():
        s_sum = s_ref[0:N_MOV_PAD] + s_ref[N_MOV_PAD:2 * N_MOV_PAD]
        w_tot = jnp.sum(s_sum)
        v = jnp.sum(me_ref[...] * s_sum[0:N_MOV][None, :], axis=1)
        vw_scr[0, 0:EMB] = v
        vw_scr[0, EMB:2 * EMB] = jnp.zeros((EMB,), jnp.float32) + w_tot
        vw_scr[0, 2 * EMB:128] = jnp.zeros((128 - 2 * EMB,), jnp.float32)

    v_col = vw_scr[0, 0:EMB][:, None]
    u = jnp.sum(ue_ref[...] * v_col, axis=0)
    out_ref[...] = u + vw_scr[0, EMB] * ub_ref[0, :]
    vw_out[...] = vw_scr[...]


def _sc_out(u_hbm, mb_hbm, vw_hbm, xv, yv, fcb,
            out_hbm,
            xidx, yidx, u_buf, mb_buf, wv_buf, fcb_buf, out_buf, sem):
    c = lax.axis_index("c")
    s = lax.axis_index("s")
    wid = s * NC + c
    ibase = wid * I_PER

    copies = []
    for i in range(I_PER // CH):
        pltpu.sync_copy(xv.at[pl.ds(ibase + i * CH, CH)], xidx.at[i])
        copies.append(pltpu.async_copy(
            u_hbm.at[xidx.at[i]], u_buf.at[pl.ds(i * CH, CH)], sem))
        pltpu.sync_copy(yv.at[pl.ds(ibase + i * CH, CH)], yidx.at[i])
        copies.append(pltpu.async_copy(
            mb_hbm.at[yidx.at[i]], mb_buf.at[pl.ds(i * CH, CH)], sem))
    pltpu.sync_copy(vw_hbm.at[0], wv_buf)
    pltpu.sync_copy(fcb, fcb_buf)
    for cp in copies:
        cp.wait()
    w_splat = wv_buf[pl.ds(EMB, L)]
    fcb_splat = fcb_buf[...]

    def body(b, carry):
        z = (u_buf[pl.ds(b * L, L)]
             + w_splat * mb_buf[pl.ds(b * L, L)] + fcb_splat)
        out_buf[pl.ds(b * L, L)] = 1.0 / (1.0 + jnp.exp(-z))
        return carry

    lax.fori_loop(0, I_PER // L, body, 0)
    pltpu.sync_copy(out_buf, out_hbm.at[pl.ds(ibase, I_PER)])


@jax.jit
def kernel(x, y, usr_embd, usr_bias, mov_embd, mov_bias, fc_W, fc_b):
    xi = x.astype(jnp.int32)
    yi = y.astype(jnp.int32)
    mesh = plsc.VectorSubcoreMesh(core_axis_name="c", subcore_axis_name="s")
    sc_params = pltpu.CompilerParams(
        needs_layout_passes=False, use_tc_tiling_on_sc=False)

    scatter = functools.partial(
        pl.kernel,
        out_type=jax.ShapeDtypeStruct((NC * N_MOV_PAD,), jnp.float32),
        mesh=mesh,
        compiler_params=sc_params,
        scratch_types=[
            pltpu.VMEM((I_PER // CH, CH), jnp.int32),
            pltpu.VMEM((I_PER // CH, CH), jnp.float32),
            pltpu.VMEM((NSEG,), jnp.float32),
            pltpu.VMEM_SHARED((N_MOV_PAD,), jnp.float32),
        ],
    )(_sc_scatter)
    s_flat = scatter(yi, fc_W.reshape(-1))

    u_all, vw = pl.pallas_call(
        _tc_user,
        grid=(GRID,),
        in_specs=[
            pl.BlockSpec((EMB, BLK), lambda i: (0, i)),
            pl.BlockSpec((1, BLK), lambda i: (0, i)),
            pl.BlockSpec((NC * N_MOV_PAD,), lambda i: (0,)),
            pl.BlockSpec((EMB, N_MOV), lambda i: (0, 0)),
        ],
        out_specs=[
            pl.BlockSpec((BLK,), lambda i: (i,)),
            pl.BlockSpec((1, 128), lambda i: (0, 0)),
        ],
        out_shape=[
            jax.ShapeDtypeStruct((N_PAD,), jnp.float32),
            jax.ShapeDtypeStruct((1, 128), jnp.float32),
        ],
        scratch_shapes=[pltpu.VMEM((1, 128), jnp.float32)],
    )(usr_embd.T, usr_bias.T, s_flat, mov_embd.T)

    final = functools.partial(
        pl.kernel,
        out_type=jax.ShapeDtypeStruct((B,), jnp.float32),
        mesh=mesh,
        compiler_params=sc_params,
        scratch_types=[
            pltpu.VMEM((I_PER // CH, CH), jnp.int32),
            pltpu.VMEM((I_PER // CH, CH), jnp.int32),
            pltpu.VMEM((I_PER,), jnp.float32),
            pltpu.VMEM((I_PER,), jnp.float32),
            pltpu.VMEM((128,), jnp.float32),
            pltpu.VMEM((L,), jnp.float32),
            pltpu.VMEM((I_PER,), jnp.float32),
            pltpu.SemaphoreType.DMA,
        ],
    )(_sc_out)
    out = final(u_all, mov_bias.reshape(-1), vw, xi, yi,
                jnp.broadcast_to(fc_b, (L,)))
    return out.reshape(B, 1)


# speedup vs baseline: 10.6924x; 1.0557x over previous
import functools

import jax
import jax.numpy as jnp
from jax import lax
from jax.experimental import pallas as pl
from jax.experimental.pallas import tpu as pltpu
from jax.experimental.pallas import tpu_sc as plsc

B = 16384
EMB = 16
L = 16
NC = 2
NS = 16
I_PER = B // (NC * NS)
CH = 128

N_USR = 1000001
N_MOV = 100001
NSEG = 6256
N_MOV_PAD = NS * NSEG
BLK = 131072
GRID = -(-N_USR // BLK)
N_PAD = GRID * BLK


def _sc_scatter(yv, fcw, s_out,
                yidx, w_buf, zero_buf, acc):
    c = lax.axis_index("c")
    s = lax.axis_index("s")
    wid = s * NC + c
    jbase = wid * I_PER

    zero = jnp.zeros((L,), jnp.float32)

    def zbody(b, carry):
        zero_buf[pl.ds(b * L, L)] = zero
        return carry

    lax.fori_loop(0, NSEG // L, zbody, 0)
    pltpu.sync_copy(zero_buf, acc.at[pl.ds(s * NSEG, NSEG)])
    plsc.subcore_barrier()

    for i in range(I_PER // CH):
        pltpu.sync_copy(yv.at[pl.ds(jbase + i * CH, CH)], yidx.at[i])
        pltpu.sync_copy(fcw.at[pl.ds(jbase + i * CH, CH)], w_buf.at[i])
        pltpu.sync_copy(w_buf.at[i], acc.at[yidx.at[i]], add=True)
    plsc.subcore_barrier()
    pltpu.sync_copy(acc.at[pl.ds(s * NSEG, NSEG)],
                    s_out.at[pl.ds(c * N_MOV_PAD + s * NSEG, NSEG)])


def _tc_user(ue_ref, ub_ref, s_ref, me_ref, out_ref, vw_out, vw_scr):
    @pl.when(pl.program_id(0) == 0)
    def _():
        s_sum = s_ref[0:N_MOV_PAD] + s_ref[N_MOV_PAD:2 * N_MOV_PAD]
        w_tot = jnp.sum(s_sum)
        v = jnp.sum(me_ref[...] * s_sum[0:N_MOV][None, :], axis=1)
        vw_scr[0, 0:EMB] = v
        vw_scr[0, EMB:2 * EMB] = jnp.zeros((EMB,), jnp.float32) + w_tot
        vw_scr[0, 2 * EMB:128] = jnp.zeros((128 - 2 * EMB,), jnp.float32)

    v_col = vw_scr[0, 0:EMB][:, None]
    u = jnp.sum(ue_ref[...] * v_col, axis=0)
    out_ref[...] = u + vw_scr[0, EMB] * ub_ref[0, :]
    vw_out[...] = vw_scr[...]


def _sc_out(u_hbm, mb_hbm, vw_hbm, xv, yv, fcb,
            out_hbm,
            xidx, yidx, u_buf, mb_buf, wv_buf, fcb_buf, out_buf, sem):
    c = lax.axis_index("c")
    s = lax.axis_index("s")
    wid = s * NC + c
    ibase = wid * I_PER

    copies = []
    for i in range(I_PER // CH):
        pltpu.sync_copy(xv.at[pl.ds(ibase + i * CH, CH)], xidx.at[i])
        copies.append(pltpu.async_copy(
            u_hbm.at[xidx.at[i]], u_buf.at[pl.ds(i * CH, CH)], sem))
        pltpu.sync_copy(yv.at[pl.ds(ibase + i * CH, CH)], yidx.at[i])
        copies.append(pltpu.async_copy(
            mb_hbm.at[yidx.at[i]], mb_buf.at[pl.ds(i * CH, CH)], sem))
    pltpu.sync_copy(vw_hbm.at[0], wv_buf)
    pltpu.sync_copy(fcb, fcb_buf)
    for cp in copies:
        cp.wait()
    w_splat = wv_buf[pl.ds(EMB, L)]
    fcb_splat = fcb_buf[...]

    def body(b, carry):
        z = (u_buf[pl.ds(b * L, L)]
             + w_splat * mb_buf[pl.ds(b * L, L)] + fcb_splat)
        out_buf[pl.ds(b * L, L)] = 1.0 / (1.0 + jnp.exp(-z))
        return carry

    lax.fori_loop(0, I_PER // L, body, 0)
    pltpu.sync_copy(out_buf, out_hbm.at[pl.ds(ibase, I_PER)])


@jax.jit
def kernel(x, y, usr_embd, usr_bias, mov_embd, mov_bias, fc_W, fc_b):
    xi = x.astype(jnp.int32)
    yi = y.astype(jnp.int32)
    mesh = plsc.VectorSubcoreMesh(core_axis_name="c", subcore_axis_name="s")
    sc_params = pltpu.CompilerParams(
        needs_layout_passes=False, use_tc_tiling_on_sc=False)

    scatter = functools.partial(
        pl.kernel,
        out_type=jax.ShapeDtypeStruct((NC * N_MOV_PAD,), jnp.float32),
        mesh=mesh,
        compiler_params=sc_params,
        scratch_types=[
            pltpu.VMEM((I_PER // CH, CH), jnp.int32),
            pltpu.VMEM((I_PER // CH, CH), jnp.float32),
            pltpu.VMEM((NSEG,), jnp.float32),
            pltpu.VMEM_SHARED((N_MOV_PAD,), jnp.float32),
        ],
    )(_sc_scatter)
    s_flat = scatter(yi, fc_W.reshape(-1))

    u_all, vw = pl.pallas_call(
        _tc_user,
        grid=(GRID,),
        in_specs=[
            pl.BlockSpec((EMB, BLK), lambda i: (0, i)),
            pl.BlockSpec((1, BLK), lambda i: (0, i)),
            pl.BlockSpec((NC * N_MOV_PAD,), lambda i: (0,)),
            pl.BlockSpec((EMB, N_MOV), lambda i: (0, 0)),
        ],
        out_specs=[
            pl.BlockSpec((BLK,), lambda i: (i,)),
            pl.BlockSpec((1, 128), lambda i: (0, 0)),
        ],
        out_shape=[
            jax.ShapeDtypeStruct((N_PAD,), jnp.float32),
            jax.ShapeDtypeStruct((1, 128), jnp.float32),
        ],
        scratch_shapes=[pltpu.VMEM((1, 128), jnp.float32)],
    )(usr_embd.T, usr_bias.T, s_flat, mov_embd.T)

    final = functools.partial(
        pl.kernel,
        out_type=jax.ShapeDtypeStruct((B,), jnp.float32),
        mesh=mesh,
        compiler_params=sc_params,
        scratch_types=[
            pltpu.VMEM((I_PER // CH, CH), jnp.int32),
            pltpu.VMEM((I_PER // CH, CH), jnp.int32),
            pltpu.VMEM((I_PER,), jnp.float32),
            pltpu.VMEM((I_PER,), jnp.float32),
            pltpu.VMEM((128,), jnp.float32),
            pltpu.VMEM((L,), jnp.float32),
            pltpu.VMEM((I_PER,), jnp.float32),
            pltpu.SemaphoreType.DMA,
        ],
    )(_sc_out)
    out = final(u_all, mov_bias.reshape(-1), vw, xi, yi,
                jnp.broadcast_to(fc_b, (L,)))
    return out.reshape(B, 1)


# speedup vs baseline: 11.5390x; 1.0792x over previous
import functools

import jax
import jax.numpy as jnp
from jax import lax
from jax.experimental import pallas as pl
from jax.experimental.pallas import tpu as pltpu
from jax.experimental.pallas import tpu_sc as plsc

B = 16384
EMB = 16
L = 16
NC = 2
NS = 16
I_PER = B // (NC * NS)
CH = 128

N_USR = 1000001
N_MOV = 100001
NSEG = 6256
N_MOV_PAD = NS * NSEG
BLK = 131072
GRID = -(-N_USR // BLK)
N_PAD = GRID * BLK


def _sc_scatter(yv, fcw, s_out,
                yidx, w_buf, zero_buf, acc):
    c = lax.axis_index("c")
    s = lax.axis_index("s")
    wid = s * NC + c
    jbase = wid * I_PER

    zero = jnp.zeros((L,), jnp.float32)

    def zbody(b, carry):
        zero_buf[pl.ds(b * L, L)] = zero
        return carry

    lax.fori_loop(0, NSEG // L, zbody, 0)
    pltpu.sync_copy(zero_buf, acc.at[pl.ds(s * NSEG, NSEG)])
    plsc.subcore_barrier()

    for i in range(I_PER // CH):
        pltpu.sync_copy(yv.at[pl.ds(jbase + i * CH, CH)], yidx.at[i])
        pltpu.sync_copy(fcw.at[pl.ds(jbase + i * CH, CH)], w_buf.at[i])
        pltpu.sync_copy(w_buf.at[i], acc.at[yidx.at[i]], add=True)
    plsc.subcore_barrier()
    pltpu.sync_copy(acc.at[pl.ds(s * NSEG, NSEG)],
                    s_out.at[pl.ds(c * N_MOV_PAD + s * NSEG, NSEG)])


def _tc_user(ue_ref, ub_ref, s_ref, me_ref, out_ref, vw_out, vw_scr):
    @pl.when(pl.program_id(0) == 0)
    def _():
        s_sum = s_ref[0:N_MOV_PAD] + s_ref[N_MOV_PAD:2 * N_MOV_PAD]
        w_tot = jnp.sum(s_sum)
        v = jnp.sum(me_ref[...] * s_sum[0:N_MOV][None, :], axis=1)
        vw_scr[0, 0:EMB] = v
        vw_scr[0, EMB:2 * EMB] = jnp.zeros((EMB,), jnp.float32) + w_tot
        vw_scr[0, 2 * EMB:128] = jnp.zeros((128 - 2 * EMB,), jnp.float32)

    u = jax.lax.dot_general(vw_scr[0:1, 0:EMB], ue_ref[...],
                            (((1,), (0,)), ((), ())))
    out_ref[...] = u[0] + vw_scr[0, EMB] * ub_ref[0, :]
    vw_out[...] = vw_scr[...]


def _sc_out(u_hbm, mb_hbm, vw_hbm, xv, yv, fcb,
            out_hbm,
            xidx, yidx, u_buf, mb_buf, wv_buf, fcb_buf, out_buf, sem):
    c = lax.axis_index("c")
    s = lax.axis_index("s")
    wid = s * NC + c
    ibase = wid * I_PER

    copies = []
    for i in range(I_PER // CH):
        pltpu.sync_copy(xv.at[pl.ds(ibase + i * CH, CH)], xidx.at[i])
        copies.append(pltpu.async_copy(
            u_hbm.at[xidx.at[i]], u_buf.at[pl.ds(i * CH, CH)], sem))
        pltpu.sync_copy(yv.at[pl.ds(ibase + i * CH, CH)], yidx.at[i])
        copies.append(pltpu.async_copy(
            mb_hbm.at[yidx.at[i]], mb_buf.at[pl.ds(i * CH, CH)], sem))
    pltpu.sync_copy(vw_hbm.at[0], wv_buf)
    pltpu.sync_copy(fcb, fcb_buf)
    for cp in copies:
        cp.wait()
    w_splat = wv_buf[pl.ds(EMB, L)]
    fcb_splat = fcb_buf[...]

    def body(b, carry):
        z = (u_buf[pl.ds(b * L, L)]
             + w_splat * mb_buf[pl.ds(b * L, L)] + fcb_splat)
        out_buf[pl.ds(b * L, L)] = 1.0 / (1.0 + jnp.exp(-z))
        return carry

    lax.fori_loop(0, I_PER // L, body, 0)
    pltpu.sync_copy(out_buf, out_hbm.at[pl.ds(ibase, I_PER)])


@jax.jit
def kernel(x, y, usr_embd, usr_bias, mov_embd, mov_bias, fc_W, fc_b):
    xi = x.astype(jnp.int32)
    yi = y.astype(jnp.int32)
    mesh = plsc.VectorSubcoreMesh(core_axis_name="c", subcore_axis_name="s")
    sc_params = pltpu.CompilerParams(
        needs_layout_passes=False, use_tc_tiling_on_sc=False)

    scatter = functools.partial(
        pl.kernel,
        out_type=jax.ShapeDtypeStruct((NC * N_MOV_PAD,), jnp.float32),
        mesh=mesh,
        compiler_params=sc_params,
        scratch_types=[
            pltpu.VMEM((I_PER // CH, CH), jnp.int32),
            pltpu.VMEM((I_PER // CH, CH), jnp.float32),
            pltpu.VMEM((NSEG,), jnp.float32),
            pltpu.VMEM_SHARED((N_MOV_PAD,), jnp.float32),
        ],
    )(_sc_scatter)
    s_flat = scatter(yi, fc_W.reshape(-1))

    u_all, vw = pl.pallas_call(
        _tc_user,
        grid=(GRID,),
        in_specs=[
            pl.BlockSpec((EMB, BLK), lambda i: (0, i)),
            pl.BlockSpec((1, BLK), lambda i: (0, i)),
            pl.BlockSpec((NC * N_MOV_PAD,), lambda i: (0,)),
            pl.BlockSpec((EMB, N_MOV), lambda i: (0, 0)),
        ],
        out_specs=[
            pl.BlockSpec((BLK,), lambda i: (i,)),
            pl.BlockSpec((1, 128), lambda i: (0, 0)),
        ],
        out_shape=[
            jax.ShapeDtypeStruct((N_PAD,), jnp.float32),
            jax.ShapeDtypeStruct((1, 128), jnp.float32),
        ],
        scratch_shapes=[pltpu.VMEM((1, 128), jnp.float32)],
    )(usr_embd.T, usr_bias.T, s_flat, mov_embd.T)

    final = functools.partial(
        pl.kernel,
        out_type=jax.ShapeDtypeStruct((B,), jnp.float32),
        mesh=mesh,
        compiler_params=sc_params,
        scratch_types=[
            pltpu.VMEM((I_PER // CH, CH), jnp.int32),
            pltpu.VMEM((I_PER // CH, CH), jnp.int32),
            pltpu.VMEM((I_PER,), jnp.float32),
            pltpu.VMEM((I_PER,), jnp.float32),
            pltpu.VMEM((128,), jnp.float32),
            pltpu.VMEM((L,), jnp.float32),
            pltpu.VMEM((I_PER,), jnp.float32),
            pltpu.SemaphoreType.DMA,
        ],
    )(_sc_out)
    out = final(u_all, mov_bias.reshape(-1), vw, xi, yi,
                jnp.broadcast_to(fc_b, (L,)))
    return out.reshape(B, 1)
